# baseline (speedup 1.0000x reference)
import jax
import jax.numpy as jnp
from jax.experimental import pallas as pl

_NS = 128
_R = 64


def _body(o_ref, d_ref, z_ref, w_ref, zall_ref, pts_ref):
    z = z_ref[...]
    w = w_ref[...]
    R = z.shape[0]

    wmid = w[:, 1:63] + 1e-5
    tri = (jax.lax.broadcasted_iota(jnp.int32, (62, 62), 0)
           <= jax.lax.broadcasted_iota(jnp.int32, (62, 62), 1)).astype(jnp.float32)
    csum = jnp.dot(wmid, tri, preferred_element_type=jnp.float32)
    cdf = csum / csum[:, 61:62]
    bins = 0.5 * (z[:, 1:] + z[:, :-1])

    zero = jnp.zeros((R, 1), jnp.float32)
    cfull = jnp.concatenate([zero, cdf], axis=1)
    cnext = jnp.concatenate([cdf, cdf[:, 61:62]], axis=1)
    bnext = jnp.concatenate([bins[:, 1:], bins[:, 62:63]], axis=1)

    slope = (bnext - bins) / jnp.maximum(cnext - cfull, 1e-30)

    jrev = jax.lax.broadcasted_iota(jnp.int32, (1, _NS), 1).astype(jnp.float32)
    u = jnp.broadcast_to((float(_NS - 1) - jrev) * (1.0 / (_NS - 1)), (R, _NS))
    acc = jnp.broadcast_to(bins[:, 0:1], (R, _NS))
    vprev = jnp.zeros((R, _NS), jnp.float32)
    for i in range(62):
        vnext = jnp.minimum(u, cfull[:, i + 1:i + 2])
        acc = acc + slope[:, i:i + 1] * (vnext - vprev)
        vprev = vnext
    samp_desc = acc

    big = jnp.full((R, 64), 3e38, jnp.float32)
    s = jnp.concatenate([z, big, samp_desc], axis=1)
    lane = jax.lax.broadcasted_iota(jnp.int32, (1, 256), 1)
    for stride in (128, 64, 32, 16, 8, 4, 2, 1):
        upper = (lane & stride) != 0
        fwd = jnp.concatenate([s[:, stride:], s[:, :stride]], axis=1)
        bwd = jnp.concatenate([s[:, 256 - stride:], s[:, :256 - stride]], axis=1)
        partner = jnp.where(upper, bwd, fwd)
        s = jnp.where(upper, jnp.maximum(s, partner), jnp.minimum(s, partner))
    zall = s[:, :192]
    zall_ref[...] = zall

    expand = (jax.lax.broadcasted_iota(jnp.int32, (192, 576), 1) // 3
              == jax.lax.broadcasted_iota(jnp.int32, (192, 576), 0)).astype(jnp.float32)
    zrep = jnp.dot(zall, expand, preferred_element_type=jnp.float32)
    mod3 = jax.lax.broadcasted_iota(jnp.int32, (1, 576), 1) % 3
    o = o_ref[...]
    d = d_ref[...]

    def rep3(a):
        a0 = jnp.broadcast_to(a[:, 0:1], (R, 576))
        a1 = jnp.broadcast_to(a[:, 1:2], (R, 576))
        a2 = jnp.broadcast_to(a[:, 2:3], (R, 576))
        return jnp.where(mod3 == 0, a0, jnp.where(mod3 == 1, a1, a2))

    pts_ref[...] = rep3(o) + rep3(d) * zrep


def kernel(rays_o, rays_d, z_vals, weights):
    N, Z = z_vals.shape
    R = _R
    zall, pts2d = pl.pallas_call(
        _body,
        grid=(N // R,),
        in_specs=[
            pl.BlockSpec((R, 3), lambda i: (i, 0)),
            pl.BlockSpec((R, 3), lambda i: (i, 0)),
            pl.BlockSpec((R, Z), lambda i: (i, 0)),
            pl.BlockSpec((R, Z), lambda i: (i, 0)),
        ],
        out_specs=[
            pl.BlockSpec((R, 192), lambda i: (i, 0)),
            pl.BlockSpec((R, 576), lambda i: (i, 0)),
        ],
        out_shape=[
            jax.ShapeDtypeStruct((N, 192), jnp.float32),
            jax.ShapeDtypeStruct((N, 576), jnp.float32),
        ],
    )(rays_o, rays_d, z_vals, weights)
    return (pts2d.reshape(N, 192, 3), zall)


# speedup vs baseline: 1.1938x; 1.1938x over previous
import jax
import jax.numpy as jnp
from jax.experimental import pallas as pl

_NS = 128
_R = 128


def _body(o_ref, d_ref, z_ref, w_ref, zall_ref, pts_ref):
    z = z_ref[...]
    w = w_ref[...]
    R = z.shape[0]

    wmid = w[:, 1:63] + 1e-5
    tri = (jax.lax.broadcasted_iota(jnp.int32, (62, 62), 0)
           <= jax.lax.broadcasted_iota(jnp.int32, (62, 62), 1)).astype(jnp.float32)
    csum = jnp.dot(wmid, tri, preferred_element_type=jnp.float32)
    cdf = csum / csum[:, 61:62]
    bins = 0.5 * (z[:, 1:] + z[:, :-1])

    zero = jnp.zeros((R, 1), jnp.float32)
    cfull = jnp.concatenate([zero, cdf], axis=1)
    cnext = jnp.concatenate([cdf, cdf[:, 61:62]], axis=1)
    bnext = jnp.concatenate([bins[:, 1:], bins[:, 62:63]], axis=1)

    slope = (bnext - bins) / jnp.maximum(cnext - cfull, 1e-30)

    jrev = jax.lax.broadcasted_iota(jnp.int32, (1, _NS), 1).astype(jnp.float32)
    u = jnp.broadcast_to((float(_NS - 1) - jrev) * (1.0 / (_NS - 1)), (R, _NS))
    acc = jnp.broadcast_to(bins[:, 0:1], (R, _NS))
    vprev = jnp.zeros((R, _NS), jnp.float32)
    for i in range(62):
        vnext = jnp.minimum(u, cfull[:, i + 1:i + 2])
        acc = acc + slope[:, i:i + 1] * (vnext - vprev)
        vprev = vnext
    samp_desc = acc

    big = jnp.full((R, 64), 3e38, jnp.float32)
    s = jnp.concatenate([z, big, samp_desc], axis=1)
    lane = jax.lax.broadcasted_iota(jnp.int32, (1, 256), 1)
    for stride in (128, 64, 32, 16, 8, 4, 2, 1):
        upper = (lane & stride) != 0
        fwd = jnp.concatenate([s[:, stride:], s[:, :stride]], axis=1)
        bwd = jnp.concatenate([s[:, 256 - stride:], s[:, :256 - stride]], axis=1)
        partner = jnp.where(upper, bwd, fwd)
        s = jnp.where(upper, jnp.maximum(s, partner), jnp.minimum(s, partner))
    zall = s[:, :192]
    zall_ref[...] = zall

    expand = (jax.lax.broadcasted_iota(jnp.int32, (192, 576), 1) // 3
              == jax.lax.broadcasted_iota(jnp.int32, (192, 576), 0)).astype(jnp.float32)
    zrep = jnp.dot(zall, expand, preferred_element_type=jnp.float32)
    mod3 = jax.lax.broadcasted_iota(jnp.int32, (1, 576), 1) % 3
    o = o_ref[...]
    d = d_ref[...]

    def rep3(a):
        a0 = jnp.broadcast_to(a[:, 0:1], (R, 576))
        a1 = jnp.broadcast_to(a[:, 1:2], (R, 576))
        a2 = jnp.broadcast_to(a[:, 2:3], (R, 576))
        return jnp.where(mod3 == 0, a0, jnp.where(mod3 == 1, a1, a2))

    pts_ref[...] = rep3(o) + rep3(d) * zrep


def kernel(rays_o, rays_d, z_vals, weights):
    N, Z = z_vals.shape
    R = _R
    zall, pts2d = pl.pallas_call(
        _body,
        grid=(N // R,),
        in_specs=[
            pl.BlockSpec((R, 3), lambda i: (i, 0)),
            pl.BlockSpec((R, 3), lambda i: (i, 0)),
            pl.BlockSpec((R, Z), lambda i: (i, 0)),
            pl.BlockSpec((R, Z), lambda i: (i, 0)),
        ],
        out_specs=[
            pl.BlockSpec((R, 192), lambda i: (i, 0)),
            pl.BlockSpec((R, 576), lambda i: (i, 0)),
        ],
        out_shape=[
            jax.ShapeDtypeStruct((N, 192), jnp.float32),
            jax.ShapeDtypeStruct((N, 576), jnp.float32),
        ],
    )(rays_o, rays_d, z_vals, weights)
    return (pts2d.reshape(N, 192, 3), zall)


# speedup vs baseline: 1.3350x; 1.1183x over previous
import jax
import jax.numpy as jnp
from jax.experimental import pallas as pl

_NS = 128
_R = 512


def _body(o_ref, d_ref, z_ref, w_ref, zall_ref, pts_ref):
    z = z_ref[...]
    w = w_ref[...]
    R = z.shape[0]

    wmid = w[:, 1:63] + 1e-5
    tri = (jax.lax.broadcasted_iota(jnp.int32, (62, 62), 0)
           <= jax.lax.broadcasted_iota(jnp.int32, (62, 62), 1)).astype(jnp.float32)
    csum = jnp.dot(wmid, tri, preferred_element_type=jnp.float32)
    cdf = csum / csum[:, 61:62]
    bins = 0.5 * (z[:, 1:] + z[:, :-1])

    zero = jnp.zeros((R, 1), jnp.float32)
    cfull = jnp.concatenate([zero, cdf], axis=1)
    cnext = jnp.concatenate([cdf, cdf[:, 61:62]], axis=1)
    bnext = jnp.concatenate([bins[:, 1:], bins[:, 62:63]], axis=1)

    slope = (bnext - bins) / jnp.maximum(cnext - cfull, 1e-30)

    jrev = jax.lax.broadcasted_iota(jnp.int32, (1, _NS), 1).astype(jnp.float32)
    u = jnp.broadcast_to((float(_NS - 1) - jrev) * (1.0 / (_NS - 1)), (R, _NS))
    acc = jnp.broadcast_to(bins[:, 0:1], (R, _NS))
    vprev = jnp.zeros((R, _NS), jnp.float32)
    for i in range(62):
        vnext = jnp.minimum(u, cfull[:, i + 1:i + 2])
        acc = acc + slope[:, i:i + 1] * (vnext - vprev)
        vprev = vnext
    samp_desc = acc

    big = jnp.full((R, 64), 3e38, jnp.float32)
    s = jnp.concatenate([z, big, samp_desc], axis=1)
    lane = jax.lax.broadcasted_iota(jnp.int32, (1, 256), 1)
    for stride in (128, 64, 32, 16, 8, 4, 2, 1):
        upper = (lane & stride) != 0
        fwd = jnp.concatenate([s[:, stride:], s[:, :stride]], axis=1)
        bwd = jnp.concatenate([s[:, 256 - stride:], s[:, :256 - stride]], axis=1)
        partner = jnp.where(upper, bwd, fwd)
        s = jnp.where(upper, jnp.maximum(s, partner), jnp.minimum(s, partner))
    zall = s[:, :192]
    zall_ref[...] = zall

    expand = (jax.lax.broadcasted_iota(jnp.int32, (192, 576), 1) // 3
              == jax.lax.broadcasted_iota(jnp.int32, (192, 576), 0)).astype(jnp.float32)
    zrep = jnp.dot(zall, expand, preferred_element_type=jnp.float32)
    mod3 = jax.lax.broadcasted_iota(jnp.int32, (1, 576), 1) % 3
    o = o_ref[...]
    d = d_ref[...]

    def rep3(a):
        a0 = jnp.broadcast_to(a[:, 0:1], (R, 576))
        a1 = jnp.broadcast_to(a[:, 1:2], (R, 576))
        a2 = jnp.broadcast_to(a[:, 2:3], (R, 576))
        return jnp.where(mod3 == 0, a0, jnp.where(mod3 == 1, a1, a2))

    pts_ref[...] = rep3(o) + rep3(d) * zrep


def kernel(rays_o, rays_d, z_vals, weights):
    N, Z = z_vals.shape
    R = _R
    zall, pts2d = pl.pallas_call(
        _body,
        grid=(N // R,),
        in_specs=[
            pl.BlockSpec((R, 3), lambda i: (i, 0)),
            pl.BlockSpec((R, 3), lambda i: (i, 0)),
            pl.BlockSpec((R, Z), lambda i: (i, 0)),
            pl.BlockSpec((R, Z), lambda i: (i, 0)),
        ],
        out_specs=[
            pl.BlockSpec((R, 192), lambda i: (i, 0)),
            pl.BlockSpec((R, 576), lambda i: (i, 0)),
        ],
        out_shape=[
            jax.ShapeDtypeStruct((N, 192), jnp.float32),
            jax.ShapeDtypeStruct((N, 576), jnp.float32),
        ],
    )(rays_o, rays_d, z_vals, weights)
    return (pts2d.reshape(N, 192, 3), zall)


# speedup vs baseline: 1.3400x; 1.0037x over previous
import jax
import jax.numpy as jnp
from jax.experimental import pallas as pl

_NS = 128
_R = 512


def _body(o_ref, d_ref, z_ref, w_ref, zall_ref, pts_ref):
    z = z_ref[...]
    w = w_ref[...]
    R = z.shape[0]

    wmid = w[:, 1:63] + 1e-5
    tri = (jax.lax.broadcasted_iota(jnp.int32, (62, 62), 0)
           <= jax.lax.broadcasted_iota(jnp.int32, (62, 62), 1)).astype(jnp.float32)
    csum = jnp.dot(wmid, tri, preferred_element_type=jnp.float32)
    cdf = csum / csum[:, 61:62]
    bins = 0.5 * (z[:, 1:] + z[:, :-1])

    zero = jnp.zeros((R, 1), jnp.float32)
    cfull = jnp.concatenate([zero, cdf], axis=1)
    cnext = jnp.concatenate([cdf, cdf[:, 61:62]], axis=1)
    bnext = jnp.concatenate([bins[:, 1:], bins[:, 62:63]], axis=1)

    slope = (bnext - bins) / jnp.maximum(cnext - cfull, 1e-30)

    _RC = 128
    jrev = jax.lax.broadcasted_iota(jnp.int32, (1, _NS), 1).astype(jnp.float32)
    u = jnp.broadcast_to((float(_NS - 1) - jrev) * (1.0 / (_NS - 1)), (_RC, _NS))
    chunks = []
    for rb in range(0, R, _RC):
        cf_c = cfull[rb:rb + _RC]
        sl_c = slope[rb:rb + _RC]
        acc = jnp.broadcast_to(bins[rb:rb + _RC, 0:1], (_RC, _NS))
        vprev = jnp.zeros((_RC, _NS), jnp.float32)
        for i in range(62):
            vnext = jnp.minimum(u, cf_c[:, i + 1:i + 2])
            acc = acc + sl_c[:, i:i + 1] * (vnext - vprev)
            vprev = vnext
        chunks.append(acc)
    samp_desc = jnp.concatenate(chunks, axis=0)

    big = jnp.full((R, 64), 3e38, jnp.float32)
    s = jnp.concatenate([z, big, samp_desc], axis=1)
    lane = jax.lax.broadcasted_iota(jnp.int32, (1, 256), 1)
    for stride in (128, 64, 32, 16, 8, 4, 2, 1):
        upper = (lane & stride) != 0
        fwd = jnp.concatenate([s[:, stride:], s[:, :stride]], axis=1)
        bwd = jnp.concatenate([s[:, 256 - stride:], s[:, :256 - stride]], axis=1)
        partner = jnp.where(upper, bwd, fwd)
        s = jnp.where(upper, jnp.maximum(s, partner), jnp.minimum(s, partner))
    zall = s[:, :192]
    zall_ref[...] = zall

    expand = (jax.lax.broadcasted_iota(jnp.int32, (192, 576), 1) // 3
              == jax.lax.broadcasted_iota(jnp.int32, (192, 576), 0)).astype(jnp.float32)
    zrep = jnp.dot(zall, expand, preferred_element_type=jnp.float32)
    mod3 = jax.lax.broadcasted_iota(jnp.int32, (1, 576), 1) % 3
    o = o_ref[...]
    d = d_ref[...]

    def rep3(a):
        a0 = jnp.broadcast_to(a[:, 0:1], (R, 576))
        a1 = jnp.broadcast_to(a[:, 1:2], (R, 576))
        a2 = jnp.broadcast_to(a[:, 2:3], (R, 576))
        return jnp.where(mod3 == 0, a0, jnp.where(mod3 == 1, a1, a2))

    pts_ref[...] = rep3(o) + rep3(d) * zrep


def kernel(rays_o, rays_d, z_vals, weights):
    N, Z = z_vals.shape
    R = _R
    zall, pts2d = pl.pallas_call(
        _body,
        grid=(N // R,),
        in_specs=[
            pl.BlockSpec((R, 3), lambda i: (i, 0)),
            pl.BlockSpec((R, 3), lambda i: (i, 0)),
            pl.BlockSpec((R, Z), lambda i: (i, 0)),
            pl.BlockSpec((R, Z), lambda i: (i, 0)),
        ],
        out_specs=[
            pl.BlockSpec((R, 192), lambda i: (i, 0)),
            pl.BlockSpec((R, 576), lambda i: (i, 0)),
        ],
        out_shape=[
            jax.ShapeDtypeStruct((N, 192), jnp.float32),
            jax.ShapeDtypeStruct((N, 576), jnp.float32),
        ],
    )(rays_o, rays_d, z_vals, weights)
    return (pts2d.reshape(N, 192, 3), zall)
